# baseline (speedup 1.0000x reference)
import functools

import jax
import jax.numpy as jnp
from jax import lax
from jax.experimental import pallas as pl
from jax.experimental.pallas import tpu as pltpu
from jax.experimental.pallas import tpu_sc as plsc

N_CAMS = 6
FH, FW, FC = 64, 176, 256
STRIDE = 4.0
N_PTS = 200 * 200 * 4
ROWS_PER_CAM = FH * FW
ZERO_ROW = N_CAMS * ROWS_PER_CAM
NQ = 200 * 200
CIN = FC * 4
COUT = 80
CPAD = 96


def _index_kernel(pm_ref, vg_ref, idx_ref):
    pts = jnp.concatenate(
        [vg_ref[...], jnp.ones((1, N_PTS), jnp.float32)], axis=0)
    t = lax.dot_general(pm_ref[...], pts, (((1,), (0,)), ((), ())))
    idxf = jnp.full((1, N_PTS), float(ZERO_ROW), jnp.float32)
    for c in range(N_CAMS):
        zu = t[3 * c:3 * c + 1, :]
        zv = t[3 * c + 1:3 * c + 2, :]
        zz = t[3 * c + 2:3 * c + 3, :]
        u = jnp.round((zu / zz) / STRIDE)
        v = jnp.round((zv / zz) / STRIDE)
        valid = ((u >= 0.0) & (v >= 0.0) & (u < float(FW)) & (v < float(FH))
                 & (zz > 0.0))
        rowf = float(c * ROWS_PER_CAM) + float(FW) * v + u
        idxf = jnp.where(valid, rowf, idxf)
    zpat = jax.lax.broadcasted_iota(jnp.int32, (1, N_PTS), 1) & 3
    idx_ref[...] = (idxf.astype(jnp.int32) << 2) | zpat


def _compute_indices(proj_flat, vg2):
    return pl.pallas_call(
        _index_kernel,
        grid=(1,),
        in_specs=[
            pl.BlockSpec((18, 4), lambda i: (0, 0)),
            pl.BlockSpec((3, N_PTS), lambda i: (0, 0)),
        ],
        out_specs=pl.BlockSpec((1, N_PTS), lambda i: (0, 0)),
        out_shape=jax.ShapeDtypeStruct((1, N_PTS), jnp.int32),
    )(proj_flat, vg2)


_TW = 512


def _table_mm_kernel(t_ref, w_ref, o_ref):
    o_ref[...] = lax.dot_general(
        t_ref[...], w_ref[...], (((1,), (0,)), ((), ())),
        preferred_element_type=jnp.float32).astype(jnp.bfloat16)


def _conv_table(table, wcat):
    return pl.pallas_call(
        _table_mm_kernel,
        grid=(ZERO_ROW // _TW,),
        in_specs=[
            pl.BlockSpec((_TW, FC), lambda i: (i, 0)),
            pl.BlockSpec((FC, 4 * CPAD), lambda i: (0, 0)),
        ],
        out_specs=pl.BlockSpec((_TW, 4 * CPAD), lambda i: (i, 0)),
        out_shape=jax.ShapeDtypeStruct((ZERO_ROW, 4 * CPAD), jnp.bfloat16),
    )(table, wcat)


_CHUNK = 128
_CHUNKS_PER_W = 40
_NBUF = 4
_ROWS_PER_W = _CHUNK * _CHUNKS_PER_W
_N_PAD = 32 * _ROWS_PER_W


def _make_gather(n_table_rows):
    mesh = plsc.VectorSubcoreMesh(core_axis_name="c", subcore_axis_name="s")

    @functools.partial(
        pl.kernel,
        mesh=mesh,
        compiler_params=pltpu.CompilerParams(use_tc_tiling_on_sc=False),
        out_type=jax.ShapeDtypeStruct((_N_PAD, CPAD), jnp.bfloat16),
        scratch_types=[
            pltpu.VMEM((_CHUNKS_PER_W, _CHUNK), jnp.int32),
            pltpu.VMEM((_NBUF, _CHUNK, CPAD), jnp.bfloat16),
        ] + [pltpu.SemaphoreType.DMA] * (2 * _NBUF),
    )
    def gather_k(idx_hbm, table_hbm, out_hbm, idx_v, rows_v, *sems):
        gsems = sems[:_NBUF]
        wsems = sems[_NBUF:]
        nc = 2
        wid = lax.axis_index("s") * nc + lax.axis_index("c")
        base = wid * _ROWS_PER_W
        pltpu.sync_copy(idx_hbm.at[pl.ds(wid * _CHUNKS_PER_W, _CHUNKS_PER_W)],
                        idx_v)

        def gather(j, b):
            return pltpu.make_async_copy(table_hbm.at[idx_v.at[j]],
                                         rows_v.at[b], gsems[b])

        def write(j, b):
            return pltpu.make_async_copy(
                rows_v.at[b], out_hbm.at[pl.ds(base + j * _CHUNK, _CHUNK)],
                wsems[b])

        for b in range(_NBUF):
            gather(b, b).start()

        def body(g, carry):
            for b in range(_NBUF):
                j = _NBUF * g + b
                gather(j, b).wait()
                write(j, b).start()
            for b in range(_NBUF):
                j = _NBUF * g + b
                jn = j + _NBUF

                @pl.when(jn < _CHUNKS_PER_W)
                def _(jn=jn, j=j, b=b):
                    write(j, b).wait()
                    gather(jn, b).start()

            return carry

        lax.fori_loop(0, _CHUNKS_PER_W // _NBUF, body, 0)
        for b in range(_NBUF):
            write(_CHUNKS_PER_W - _NBUF + b, b).wait()

    return gather_k


_TQ = 1000


def _sumz_kernel(v_ref, y_ref, s_ref):
    v = v_ref[...].astype(jnp.float32).reshape(_TQ, 4, CPAD)
    yb = jnp.sum(v, axis=1)
    y_ref[...] = yb
    ssum = jnp.sum(yb, axis=0, keepdims=True)
    ssq = jnp.sum(yb * yb, axis=0, keepdims=True)

    @pl.when(pl.program_id(0) == 0)
    def _():
        s_ref[...] = jnp.zeros_like(s_ref)

    s_ref[...] += jnp.concatenate([ssum, ssq], axis=0)


def _sum_planes(vol4):
    return pl.pallas_call(
        _sumz_kernel,
        grid=(NQ // _TQ,),
        in_specs=[pl.BlockSpec((4 * _TQ, CPAD), lambda i: (i, 0))],
        out_specs=[
            pl.BlockSpec((_TQ, CPAD), lambda i: (i, 0)),
            pl.BlockSpec((2, CPAD), lambda i: (0, 0)),
        ],
        out_shape=[
            jax.ShapeDtypeStruct((NQ, CPAD), jnp.float32),
            jax.ShapeDtypeStruct((2, CPAD), jnp.float32),
        ],
    )(vol4)


def _bn_kernel(y_ref, s_ref, g_ref, b_ref, o_ref):
    s = s_ref[...]
    inv_n = 1.0 / float(NQ)
    mean = s[0:1] * inv_n
    var = s[1:2] * inv_n - mean * mean
    scale = g_ref[...] * lax.rsqrt(var + 1e-5)
    shift = b_ref[...] - mean * scale
    o_ref[...] = jnp.maximum(y_ref[...] * scale + shift, 0.0)


def _bn_relu(y, s, gamma, beta):
    return pl.pallas_call(
        _bn_kernel,
        grid=(NQ // _TQ,),
        in_specs=[
            pl.BlockSpec((_TQ, CPAD), lambda i: (i, 0)),
            pl.BlockSpec((2, CPAD), lambda i: (0, 0)),
            pl.BlockSpec((1, CPAD), lambda i: (0, 0)),
            pl.BlockSpec((1, CPAD), lambda i: (0, 0)),
        ],
        out_specs=pl.BlockSpec((_TQ, CPAD), lambda i: (i, 0)),
        out_shape=jax.ShapeDtypeStruct((NQ, CPAD), jnp.float32),
    )(y, s, gamma, beta)


def kernel(img_features, lidar2image, img_aug_matrix, lidar_aug_matrix,
           point_clouds, camera2ego, lidar2ego, cam_intrinsic, cam_2_lidar,
           img_metas, voxel_grid, conv_w, conv_b, bn_gamma, bn_beta):
    del point_clouds, camera2ego, lidar2ego, cam_intrinsic, cam_2_lidar
    del img_metas, conv_b
    feats = img_features[0]
    proj = (img_aug_matrix[0] @ lidar2image[0] @ lidar_aug_matrix[0])[:, :3, :]
    proj_flat = proj.reshape(18, 4)

    table = jnp.transpose(feats, (0, 2, 3, 1)).reshape(ZERO_ROW, FC)
    wcat = jnp.pad(conv_w.reshape(COUT, 4, FC),
                   ((0, CPAD - COUT), (0, 0), (0, 0))).transpose(
                       2, 1, 0).reshape(FC, 4 * CPAD)
    t4 = jnp.concatenate(
        [_conv_table(table, wcat).reshape(4 * ZERO_ROW, CPAD),
         jnp.zeros((4, CPAD), jnp.bfloat16)], axis=0)

    vg2 = voxel_grid.reshape(3, N_PTS)
    idx = _compute_indices(proj_flat, vg2)
    idx2 = jnp.concatenate(
        [idx.reshape(N_PTS // _CHUNK, _CHUNK),
         jnp.full(((_N_PAD - N_PTS) // _CHUNK, _CHUNK), 4 * ZERO_ROW,
                  jnp.int32)], axis=0)

    vol = _make_gather(t4.shape[0])(idx2, t4)

    y, s = _sum_planes(vol[:N_PTS])
    gpad = jnp.pad(bn_gamma, (0, CPAD - COUT)).reshape(1, CPAD)
    bpad = jnp.pad(bn_beta, (0, CPAD - COUT)).reshape(1, CPAD)
    out = _bn_relu(y, s, gpad, bpad)
    return jnp.transpose(out)[:COUT].reshape(1, COUT, 200, 200)


# speedup vs baseline: 1.0939x; 1.0939x over previous
import functools

import jax
import jax.numpy as jnp
from jax import lax
from jax.experimental import pallas as pl
from jax.experimental.pallas import tpu as pltpu
from jax.experimental.pallas import tpu_sc as plsc

N_CAMS = 6
FH, FW, FC = 64, 176, 256
STRIDE = 4.0
N_PTS = 200 * 200 * 4
ROWS_PER_CAM = FH * FW
ZERO_ROW = N_CAMS * ROWS_PER_CAM
NQ = 200 * 200
CIN = FC * 4
COUT = 80
CPAD = 128
ROWS_PLANE = ZERO_ROW + 512


def _index_kernel(pm_ref, vg_ref, idx_ref):
    pts = jnp.concatenate(
        [vg_ref[...], jnp.ones((1, N_PTS), jnp.float32)], axis=0)
    t = lax.dot_general(pm_ref[...], pts, (((1,), (0,)), ((), ())))
    idxf = jnp.full((1, N_PTS), float(ZERO_ROW), jnp.float32)
    for c in range(N_CAMS):
        zu = t[3 * c:3 * c + 1, :]
        zv = t[3 * c + 1:3 * c + 2, :]
        zz = t[3 * c + 2:3 * c + 3, :]
        u = jnp.round((zu / zz) / STRIDE)
        v = jnp.round((zv / zz) / STRIDE)
        valid = ((u >= 0.0) & (v >= 0.0) & (u < float(FW)) & (v < float(FH))
                 & (zz > 0.0))
        rowf = float(c * ROWS_PER_CAM) + float(FW) * v + u
        idxf = jnp.where(valid, rowf, idxf)
    zpat = jax.lax.broadcasted_iota(jnp.int32, (1, N_PTS), 1) & 3
    idx_ref[...] = idxf.astype(jnp.int32) + zpat * ROWS_PLANE


def _compute_indices(proj_flat, vg2):
    return pl.pallas_call(
        _index_kernel,
        grid=(1,),
        in_specs=[
            pl.BlockSpec((18, 4), lambda i: (0, 0)),
            pl.BlockSpec((3, N_PTS), lambda i: (0, 0)),
        ],
        out_specs=pl.BlockSpec((1, N_PTS), lambda i: (0, 0)),
        out_shape=jax.ShapeDtypeStruct((1, N_PTS), jnp.int32),
    )(proj_flat, vg2)


_TW = 512


def _table_mm_kernel(t_ref, w_ref, o_ref):
    o_ref[0] = lax.dot_general(t_ref[...], w_ref[0],
                               (((1,), (0,)), ((), ())),
                               preferred_element_type=jnp.float32)


def _conv_table(table, wz):
    return pl.pallas_call(
        _table_mm_kernel,
        grid=(4, ROWS_PLANE // _TW),
        in_specs=[
            pl.BlockSpec((_TW, FC), lambda z, i: (i, 0)),
            pl.BlockSpec((1, FC, CPAD), lambda z, i: (z, 0, 0)),
        ],
        out_specs=pl.BlockSpec((1, _TW, CPAD), lambda z, i: (z, i, 0)),
        out_shape=jax.ShapeDtypeStruct((4, ROWS_PLANE, CPAD), jnp.float32),
    )(table, wz)


_CHUNK = 128
_CHUNKS_PER_W = 40
_NBUF = 4
_ROWS_PER_W = _CHUNK * _CHUNKS_PER_W
_N_PAD = 32 * _ROWS_PER_W


def _make_gather(n_table_rows):
    mesh = plsc.VectorSubcoreMesh(core_axis_name="c", subcore_axis_name="s")

    @functools.partial(
        pl.kernel,
        mesh=mesh,
        out_type=jax.ShapeDtypeStruct((_N_PAD, CPAD), jnp.float32),
        scratch_types=[
            pltpu.VMEM((_CHUNKS_PER_W, _CHUNK), jnp.int32),
            pltpu.VMEM((_NBUF, _CHUNK, CPAD), jnp.float32),
        ] + [pltpu.SemaphoreType.DMA] * (2 * _NBUF),
    )
    def gather_k(idx_hbm, table_hbm, out_hbm, idx_v, rows_v, *sems):
        gsems = sems[:_NBUF]
        wsems = sems[_NBUF:]
        nc = 2
        wid = lax.axis_index("s") * nc + lax.axis_index("c")
        base = wid * _ROWS_PER_W
        pltpu.sync_copy(idx_hbm.at[pl.ds(wid * _CHUNKS_PER_W, _CHUNKS_PER_W)],
                        idx_v)

        def gather(j, b):
            return pltpu.make_async_copy(table_hbm.at[idx_v.at[j]],
                                         rows_v.at[b], gsems[b])

        def write(j, b):
            return pltpu.make_async_copy(
                rows_v.at[b], out_hbm.at[pl.ds(base + j * _CHUNK, _CHUNK)],
                wsems[b])

        for b in range(_NBUF):
            gather(b, b).start()

        def body(g, carry):
            for b in range(_NBUF):
                j = _NBUF * g + b
                gather(j, b).wait()
                write(j, b).start()
            for b in range(_NBUF):
                j = _NBUF * g + b
                jn = j + _NBUF

                @pl.when(jn < _CHUNKS_PER_W)
                def _(jn=jn, j=j, b=b):
                    write(j, b).wait()
                    gather(jn, b).start()

            return carry

        lax.fori_loop(0, _CHUNKS_PER_W // _NBUF, body, 0)
        for b in range(_NBUF):
            write(_CHUNKS_PER_W - _NBUF + b, b).wait()

    return gather_k


_TQ = 1000


def _sumz_kernel(v_ref, y_ref, s_ref):
    v = v_ref[...].reshape(_TQ, 4, CPAD)
    yb = jnp.sum(v, axis=1)
    y_ref[...] = yb
    ssum = jnp.sum(yb, axis=0, keepdims=True)
    ssq = jnp.sum(yb * yb, axis=0, keepdims=True)

    @pl.when(pl.program_id(0) == 0)
    def _():
        s_ref[...] = jnp.zeros_like(s_ref)

    s_ref[...] += jnp.concatenate([ssum, ssq], axis=0)


def _sum_planes(vol4):
    return pl.pallas_call(
        _sumz_kernel,
        grid=(NQ // _TQ,),
        in_specs=[pl.BlockSpec((4 * _TQ, CPAD), lambda i: (i, 0))],
        out_specs=[
            pl.BlockSpec((_TQ, CPAD), lambda i: (i, 0)),
            pl.BlockSpec((2, CPAD), lambda i: (0, 0)),
        ],
        out_shape=[
            jax.ShapeDtypeStruct((NQ, CPAD), jnp.float32),
            jax.ShapeDtypeStruct((2, CPAD), jnp.float32),
        ],
    )(vol4)


def _bn_kernel(y_ref, s_ref, g_ref, b_ref, o_ref):
    s = s_ref[...]
    inv_n = 1.0 / float(NQ)
    mean = s[0:1] * inv_n
    var = s[1:2] * inv_n - mean * mean
    scale = g_ref[...] * lax.rsqrt(var + 1e-5)
    shift = b_ref[...] - mean * scale
    o_ref[...] = jnp.maximum(y_ref[...] * scale + shift, 0.0)


def _bn_relu(y, s, gamma, beta):
    return pl.pallas_call(
        _bn_kernel,
        grid=(NQ // _TQ,),
        in_specs=[
            pl.BlockSpec((_TQ, CPAD), lambda i: (i, 0)),
            pl.BlockSpec((2, CPAD), lambda i: (0, 0)),
            pl.BlockSpec((1, CPAD), lambda i: (0, 0)),
            pl.BlockSpec((1, CPAD), lambda i: (0, 0)),
        ],
        out_specs=pl.BlockSpec((_TQ, CPAD), lambda i: (i, 0)),
        out_shape=jax.ShapeDtypeStruct((NQ, CPAD), jnp.float32),
    )(y, s, gamma, beta)


def kernel(img_features, lidar2image, img_aug_matrix, lidar_aug_matrix,
           point_clouds, camera2ego, lidar2ego, cam_intrinsic, cam_2_lidar,
           img_metas, voxel_grid, conv_w, conv_b, bn_gamma, bn_beta):
    del point_clouds, camera2ego, lidar2ego, cam_intrinsic, cam_2_lidar
    del img_metas, conv_b
    feats = img_features[0]
    proj = (img_aug_matrix[0] @ lidar2image[0] @ lidar_aug_matrix[0])[:, :3, :]
    proj_flat = proj.reshape(18, 4)

    table = jnp.pad(jnp.transpose(feats, (0, 2, 3, 1)).reshape(ZERO_ROW, FC),
                    ((0, ROWS_PLANE - ZERO_ROW), (0, 0)))
    wz = jnp.pad(conv_w.reshape(COUT, 4, FC),
                 ((0, CPAD - COUT), (0, 0), (0, 0))).transpose(1, 2, 0)
    t4 = _conv_table(table, wz).reshape(4 * ROWS_PLANE, CPAD)

    vg2 = voxel_grid.reshape(3, N_PTS)
    idx = _compute_indices(proj_flat, vg2)
    idx2 = jnp.concatenate(
        [idx.reshape(N_PTS // _CHUNK, _CHUNK),
         jnp.full(((_N_PAD - N_PTS) // _CHUNK, _CHUNK), ZERO_ROW,
                  jnp.int32)], axis=0)

    vol = _make_gather(t4.shape[0])(idx2, t4)

    y, s = _sum_planes(vol[:N_PTS])
    gpad = jnp.pad(bn_gamma, (0, CPAD - COUT)).reshape(1, CPAD)
    bpad = jnp.pad(bn_beta, (0, CPAD - COUT)).reshape(1, CPAD)
    out = _bn_relu(y, s, gpad, bpad)
    return jnp.transpose(out)[:COUT].reshape(1, COUT, 200, 200)
